# baseline (speedup 1.0000x reference)
import functools

import jax
import jax.numpy as jnp
from jax import lax
from jax.experimental import pallas as pl
from jax.experimental.pallas import tpu as pltpu
from jax.experimental.pallas import tpu_sc as plsc

HIGH = lax.Precision.HIGHEST
EPS = 1e-5

NC = 2
NS = 16
CH = 128


def _layer_norm(h, g, b):
    m = jnp.mean(h, axis=-1, keepdims=True)
    v = jnp.mean((h - m) ** 2, axis=-1, keepdims=True)
    return (h - m) / jnp.sqrt(v + EPS) * g + b


def _encoder_call(x, enc_W, enc_b, enc_g, enc_be, W1a0):
    n, c = x.shape
    h_dim = enc_W.shape[1]
    p_dim = W1a0.shape[1]
    bn = 1000

    def body(x_ref, w_ref, b_ref, g_ref, be_ref, wp_ref, h_ref, p_ref):
        h = jnp.dot(x_ref[...], w_ref[...], precision=HIGH) + b_ref[...]
        h = _layer_norm(h, g_ref[...], be_ref[...])
        h = jnp.maximum(h, 0.0)
        h_ref[...] = h
        p_ref[...] = jnp.dot(h, wp_ref[...], precision=HIGH)

    return pl.pallas_call(
        body,
        grid=(n // bn,),
        in_specs=[
            pl.BlockSpec((bn, c), lambda i: (i, 0)),
            pl.BlockSpec((c, h_dim), lambda i: (0, 0)),
            pl.BlockSpec((1, h_dim), lambda i: (0, 0)),
            pl.BlockSpec((1, h_dim), lambda i: (0, 0)),
            pl.BlockSpec((1, h_dim), lambda i: (0, 0)),
            pl.BlockSpec((h_dim, p_dim), lambda i: (0, 0)),
        ],
        out_specs=[
            pl.BlockSpec((bn, h_dim), lambda i: (i, 0)),
            pl.BlockSpec((bn, p_dim), lambda i: (i, 0)),
        ],
        out_shape=[
            jax.ShapeDtypeStruct((n, h_dim), jnp.float32),
            jax.ShapeDtypeStruct((n, p_dim), jnp.float32),
        ],
    )(x, enc_W, enc_b, enc_g, enc_be, W1a0)


def _q_call(ea_pad, e_real, w1bs, b1s):
    e_pad, a = ea_pad.shape
    p_dim = w1bs[0].shape[1]
    be = 4096
    nl = len(w1bs)

    def body(ea_ref, *refs):
        w_refs = refs[:nl]
        b_refs = refs[nl:2 * nl]
        out_refs = refs[2 * nl:]
        i = pl.program_id(0)
        rows = i * be + lax.broadcasted_iota(jnp.int32, (be, 1), 0)
        valid = rows < e_real
        ea = ea_ref[...]
        neg = jnp.full((be, p_dim), -1e30, jnp.float32)
        for l in range(nl):
            q = jnp.dot(ea, w_refs[l][...], precision=HIGH) + b_refs[l][...]
            out_refs[l][...] = jnp.where(valid, q, neg)

    return pl.pallas_call(
        body,
        grid=(e_pad // be,),
        in_specs=[pl.BlockSpec((be, a), lambda i: (i, 0))]
        + [pl.BlockSpec((a, p_dim), lambda i: (0, 0))] * nl
        + [pl.BlockSpec((1, p_dim), lambda i: (0, 0))] * nl,
        out_specs=[pl.BlockSpec((be, p_dim), lambda i: (i, 0))] * nl,
        out_shape=[jax.ShapeDtypeStruct((e_pad, p_dim), jnp.float32)] * nl,
    )(ea_pad, *w1bs, *b1s)


def _post_call(h, acc, cnt, half, W2, b2, g, be, Wp):
    n, h_dim = h.shape
    nt = acc.shape[1]
    ct = cnt.shape[1]
    p_dim2 = W2.shape[0]
    bn = 1000
    hb = half // bn
    with_p = Wp is not None

    def body(h_ref, a0_ref, a1_ref, c_ref, w2_ref, b2_ref,
             g_ref, be_ref, *rest):
        if with_p:
            wp_ref, h_out, p_out = rest
        else:
            (h_out,) = rest
        acc_b = a0_ref[0] + a1_ref[0]
        cnt_b = c_ref[0, :, 0:1]
        cntc = jnp.maximum(cnt_b, 1.0)
        hn = jnp.dot(acc_b, w2_ref[...], precision=HIGH) / cntc \
            + b2_ref[...] * jnp.minimum(cnt_b, 1.0)
        h2 = _layer_norm(h_ref[...] + hn, g_ref[...], be_ref[...])
        h_out[...] = h2
        if with_p:
            p_out[...] = jnp.dot(h2, wp_ref[...], precision=HIGH)

    in_specs = [
        pl.BlockSpec((bn, h_dim), lambda i: (i, 0)),
        pl.BlockSpec((1, bn, p_dim2), lambda i: (0, i, 0)),
        pl.BlockSpec((1, bn, p_dim2), lambda i: (1, i, 0)),
        pl.BlockSpec((1, bn, 16), lambda i: (i // hb, i - (i // hb) * hb, 0)),
        pl.BlockSpec((p_dim2, h_dim), lambda i: (0, 0)),
        pl.BlockSpec((1, h_dim), lambda i: (0, 0)),
        pl.BlockSpec((1, h_dim), lambda i: (0, 0)),
        pl.BlockSpec((1, h_dim), lambda i: (0, 0)),
    ]
    out_specs = [pl.BlockSpec((bn, h_dim), lambda i: (i, 0))]
    out_shape = [jax.ShapeDtypeStruct((n, h_dim), jnp.float32)]
    args = [h, acc, acc, cnt, W2, b2, g, be]
    if with_p:
        pd = Wp.shape[1]
        in_specs.append(pl.BlockSpec((h_dim, pd), lambda i: (0, 0)))
        out_specs.append(pl.BlockSpec((bn, pd), lambda i: (i, 0)))
        out_shape.append(jax.ShapeDtypeStruct((n, pd), jnp.float32))
        args.append(Wp)
    res = pl.pallas_call(
        body,
        grid=(n // bn,),
        in_specs=in_specs,
        out_specs=out_specs,
        out_shape=out_shape,
    )(*args)
    return res if with_p else (res[0], None)


def _heads_call(h, dW1, db1, dW2, db2, vW1, vb1, vW2, vb2):
    n, h_dim = h.shape
    hh = dW1.shape[1]
    bn = 1000

    def body(h_ref, dw1, db1_, dw2, db2_, vw1, vb1_, vw2, vb2_,
             d_ref, v_ref):
        h = h_ref[...]
        dh = jnp.maximum(jnp.dot(h, dw1[...], precision=HIGH) + db1_[...], 0.0)
        d_ref[...] = jnp.dot(dh, dw2[...], precision=HIGH) + db2_[...]
        vh = jnp.maximum(jnp.dot(h, vw1[...], precision=HIGH) + vb1_[...], 0.0)
        v_ref[...] = jnp.dot(vh, vw2[...], precision=HIGH) + vb2_[...]

    return pl.pallas_call(
        body,
        grid=(n // bn,),
        in_specs=[
            pl.BlockSpec((bn, h_dim), lambda i: (i, 0)),
            pl.BlockSpec((h_dim, hh), lambda i: (0, 0)),
            pl.BlockSpec((1, hh), lambda i: (0, 0)),
            pl.BlockSpec((hh, 1), lambda i: (0, 0)),
            pl.BlockSpec((1, 1), lambda i: (0, 0)),
            pl.BlockSpec((h_dim, hh), lambda i: (0, 0)),
            pl.BlockSpec((1, hh), lambda i: (0, 0)),
            pl.BlockSpec((hh, 2), lambda i: (0, 0)),
            pl.BlockSpec((1, 2), lambda i: (0, 0)),
        ],
        out_specs=[
            pl.BlockSpec((bn, 1), lambda i: (i, 0)),
            pl.BlockSpec((bn, 2), lambda i: (i, 0)),
        ],
        out_shape=[
            jax.ShapeDtypeStruct((n, 1), jnp.float32),
            jax.ShapeDtypeStruct((n, 2), jnp.float32),
        ],
    )(h, dW1, db1, dW2, db2, vW1, vb1, vW2, vb2)


def _edge_gather_sc_call(p, srcp, qp):
    n_nodes, hh = p.shape
    e_pad = srcp.shape[0]
    e_per_w = e_pad // (NC * NS)
    mesh = plsc.VectorSubcoreMesh(core_axis_name="c", subcore_axis_name="s")

    @functools.partial(
        pl.kernel,
        out_type=jax.ShapeDtypeStruct((e_pad, hh), jnp.float32),
        mesh=mesh,
        compiler_params=pltpu.CompilerParams(use_tc_tiling_on_sc=False),
        scratch_types=[
            pltpu.VMEM((CH,), jnp.int32),
            pltpu.VMEM((CH, hh), jnp.float32),
            pltpu.VMEM((CH, hh), jnp.float32),
            pltpu.SemaphoreType.DMA,
        ],
    )
    def gather_kernel(p_hbm, src_hbm, q_hbm, out_hbm, srcv, gbuf, qv, sem):
        cid = lax.axis_index("c")
        sid = lax.axis_index("s")
        base = (cid * NS + sid) * e_per_w

        @pl.loop(0, e_per_w, step=CH)
        def _(t):
            b = base + t
            pltpu.sync_copy(src_hbm.at[pl.ds(b, CH)], srcv)
            pltpu.sync_copy(q_hbm.at[pl.ds(b, CH)], qv)
            pltpu.async_copy(p_hbm.at[srcv], gbuf, sem).wait()

            @pl.loop(0, CH)
            def _(i):
                for c in range(0, hh, 16):
                    v = gbuf[i, pl.ds(c, 16)] + qv[i, pl.ds(c, 16)]
                    gbuf[i, pl.ds(c, 16)] = jnp.maximum(v, 0.0)

            pltpu.sync_copy(gbuf, out_hbm.at[pl.ds(b, CH)])

    return gather_kernel(p, srcp, qp)


def kernel(x, edge_index, edge_attr, params):
    n, _ = x.shape
    e = edge_index.shape[1]
    h_dim = params['enc_W'].shape[1]

    e_pad = ((e + NC * NS * CH - 1) // (NC * NS * CH)) * (NC * NS * CH)
    nt = ((n + 1 + NS * 40 - 1) // (NS * 40)) * (NS * 40)
    half = n // 2
    ct = ((half + 1 + NS * 40 - 1) // (NS * 40)) * (NS * 40)

    src = edge_index[0]
    dst = edge_index[1]
    pad = e_pad - e
    srcp = jnp.concatenate([src, jnp.zeros((pad,), jnp.int32)])
    dstp = jnp.concatenate([dst, jnp.full((pad,), n, jnp.int32)])
    ea_pad = jnp.concatenate(
        [edge_attr, jnp.zeros((pad, edge_attr.shape[1]), jnp.float32)])

    def r2(v):
        return v.reshape(1, -1)

    mp = params['mp']
    w1as = [lp['W1'][:h_dim] for lp in mp]
    w1bs = [lp['W1'][h_dim:] for lp in mp]
    b1s = [r2(lp['b1']) for lp in mp]

    qs = _q_call(ea_pad, e, w1bs, b1s)

    ones_ = jnp.ones((e_pad, 16), jnp.float32)
    cs = []
    for c in range(NC):
        d_ = dstp - c * half
        d_ = jnp.where((d_ >= 0) & (d_ < half), d_, half)
        cs.append(jax.ops.segment_sum(ones_, d_, num_segments=ct))
    cnt = jnp.stack(cs)

    h, pcur = _encoder_call(x, params['enc_W'], r2(params['enc_b']),
                            r2(params['enc_g']), r2(params['enc_be']),
                            w1as[0])

    hn_ = e_pad // 2
    for l, lp in enumerate(mp):
        r = _edge_gather_sc_call(pcur, srcp, qs[l])
        acc = jnp.stack([
            jax.ops.segment_sum(r[:hn_], dstp[:hn_], num_segments=nt),
            jax.ops.segment_sum(r[hn_:], dstp[hn_:], num_segments=nt)])
        wp_next = w1as[l + 1] if l + 1 < len(mp) else None
        h, pcur = _post_call(h, acc, cnt, half, lp['W2'], r2(lp['b2']),
                             r2(lp['g']), r2(lp['be']), wp_next)

    depth, velocity = _heads_call(
        h, params['d_W1'], r2(params['d_b1']), params['d_W2'],
        r2(params['d_b2']), params['v_W1'], r2(params['v_b1']),
        params['v_W2'], r2(params['v_b2']))
    return depth, velocity
